# baseline (speedup 1.0000x reference)
import functools

import jax
import jax.numpy as jnp
import numpy as np
from jax import lax
from jax.experimental import pallas as pl
from jax.experimental.pallas import tpu as pltpu
from jax.experimental.pallas import tpu_sc as plsc

_E_IDS = np.array([0, 100, 200, 300, 400, 500, 600, 700, 800, 900, 1000, 1100,
                   1200, 1300, 1400, 1500, 1600, 1700, 1800, 1900, 2000, 2100,
                   2200, 2300, 2400, 2500, 2600, 2700, 2800, 2900, 3000, 3100,
                   3200, 3300, 3400, 3500, 3600, 3700, 3800, 3900], dtype=np.int64)
_E_ANG = np.array([0, 9, 18, 27, 36, 45, 54, 63, 72, 81, 90, 99, 108, 117, 126,
                   135, 144, 153, 162, 171, 180, 189, 198, 207, 216, 225, 234,
                   243, 252, 261, 270, 279, 288, 297, 306, 315, 324, 333, 342,
                   351], dtype=np.float32)
_I_IDS = np.array([50, 150, 250, 350, 450, 550, 650, 750, 850, 950, 1050, 1150,
                   1250, 1350, 1450, 1550, 1650, 1750, 1850, 1950, 2050, 2150,
                   2250, 2350, 2450, 2550, 2650, 2750, 2850, 2950, 3050, 3150,
                   3250, 3350, 3450, 3550, 3650, 3750, 3850, 3950], dtype=np.int64)
_I_ANG = np.array([4, 13, 22, 31, 40, 49, 58, 67, 76, 85, 94, 103, 112, 121,
                   130, 139, 148, 157, 166, 175, 184, 193, 202, 211, 220, 229,
                   238, 247, 256, 265, 274, 283, 292, 301, 310, 319, 328, 337,
                   346, 355], dtype=np.float32)
_MAIN_ANGLES = np.array([0, 45, 90, 135, 180, 225, 270, 315], dtype=np.float32)

_N = 4096
_ROWS = 8 * 2048
_NW = 32
_RPW = _ROWS // _NW
_K = 80
_L = 16
_GRP = 4


def _build_w() -> np.ndarray:
    rows = []
    for ids, ang in ((_E_IDS, _E_ANG), (_I_IDS, _I_ANG)):
        diff = np.abs(ang[:, None] - _MAIN_ANGLES[None, :])
        min_idx = np.argmin(diff, axis=1)
        closest = _MAIN_ANGLES[min_idx]
        order = np.argsort(closest, kind="stable")
        unique_angles, inv = np.unique(closest[order], return_inverse=True)
        nseg = int(unique_angles.shape[0])
        cnt = np.bincount(inv, minlength=nseg).astype(np.float32)
        sorted_ids = ids[order]
        for b in range(nseg):
            w = np.zeros(_N, np.float32)
            w[sorted_ids[inv == b]] += 1.0 / cnt[b]
            w[ids] -= 1.0 / float(ids.shape[0])
            rows.append(w)
    return np.stack(rows).astype(np.float32)


def _build_consts():
    w = _build_w()
    cols = np.arange(_K, dtype=np.int64) * 50
    g_off = (cols // _L).astype(np.int64)
    lane = (cols % _L).astype(np.int64)
    wid = np.arange(_NW, dtype=np.int64)[:, None, None]
    rl = np.arange(_RPW, dtype=np.int64)[None, None, :]
    idx = (wid * _RPW + rl) * (_N // _L) + g_off[None, :, None]
    idx = idx.reshape(_NW, (_K * _RPW) // 128, 128).astype(np.int32)
    m2 = np.zeros((16, _K, _L), np.float32)
    for k in range(_K):
        m2[:, k, lane[k]] = w[:, cols[k]]
    return idx, m2.reshape(16, _K * _L)


_IDX_NP, _M2_NP = _build_consts()


def _sc_partial_body(table_hbm, idx_hbm, out_hbm, idx_v, dbuf, acc, sem):
    wid = lax.axis_index("s") * 2 + lax.axis_index("c")
    pltpu.sync_copy(idx_hbm.at[wid], idx_v)
    n_chunks = _GRP * _K

    def _gather(c, buf):
        return pltpu.make_async_copy(table_hbm.at[idx_v.at[c]], dbuf.at[buf], sem)

    _gather(0, 0).start()

    def _step(k, carry):
        total = None
        for j in range(_GRP):
            c = k * _GRP + j
            h = j % 2
            _gather(c, h).wait()

            @pl.when(c < n_chunks - 1)
            def _fire():
                _gather(c + 1, (j + 1) % 2).start()

            vals = []
            for g in range(8):
                s = dbuf[h, 16 * g]
                for i in range(1, 16):
                    s = s + dbuf[h, 16 * g + i]
                vals.append(s)
            while len(vals) > 1:
                vals = [vals[m] + vals[m + 1] for m in range(0, len(vals), 2)]
            total = vals[0] if total is None else total + vals[0]
        acc[k] = total
        return carry

    lax.fori_loop(0, _K, _step, 0)
    pltpu.sync_copy(acc, out_hbm.at[wid])


@functools.lru_cache(maxsize=1)
def _sc_partial():
    return pl.kernel(
        _sc_partial_body,
        mesh=plsc.VectorSubcoreMesh(core_axis_name="c", subcore_axis_name="s"),
        out_type=jax.ShapeDtypeStruct((_NW, _K, _L), jnp.float32),
        scratch_types=[
            pltpu.VMEM((_IDX_NP.shape[1], 128), jnp.int32),
            pltpu.VMEM((2, 128, _L), jnp.float32),
            pltpu.VMEM((_K, _L), jnp.float32),
            pltpu.SemaphoreType.DMA,
        ],
        compiler_params=pltpu.CompilerParams(use_tc_tiling_on_sc=False),
    )


def _tc_finish_body(p_ref, m_ref, out_ref):
    s = p_ref[...].sum(axis=0, keepdims=True)
    q = (m_ref[...] * s).sum(axis=1, keepdims=True)
    scale = 1.0 / (float(_ROWS) * float(_ROWS) * 8.0)
    out_ref[...] = (jnp.sum(q * q, keepdims=True) * scale).reshape(1, 1)


@jax.jit
def kernel(_spikes):
    table = _spikes.reshape(_ROWS * (_N // _L), _L)
    partial = _sc_partial()(table, jnp.asarray(_IDX_NP))
    loss = pl.pallas_call(
        _tc_finish_body,
        in_specs=[
            pl.BlockSpec((_NW, _K * _L), lambda: (0, 0)),
            pl.BlockSpec((16, _K * _L), lambda: (0, 0)),
        ],
        out_specs=pl.BlockSpec((1, 1), lambda: (0, 0)),
        out_shape=jax.ShapeDtypeStruct((1, 1), jnp.float32),
    )(partial.reshape(_NW, _K * _L), jnp.asarray(_M2_NP))
    return loss[0, 0]


# speedup vs baseline: 1.0260x; 1.0260x over previous
import functools

import jax
import jax.numpy as jnp
import numpy as np
from jax import lax
from jax.experimental import pallas as pl
from jax.experimental.pallas import tpu as pltpu
from jax.experimental.pallas import tpu_sc as plsc

_E_IDS = np.array([0, 100, 200, 300, 400, 500, 600, 700, 800, 900, 1000, 1100,
                   1200, 1300, 1400, 1500, 1600, 1700, 1800, 1900, 2000, 2100,
                   2200, 2300, 2400, 2500, 2600, 2700, 2800, 2900, 3000, 3100,
                   3200, 3300, 3400, 3500, 3600, 3700, 3800, 3900], dtype=np.int64)
_E_ANG = np.array([0, 9, 18, 27, 36, 45, 54, 63, 72, 81, 90, 99, 108, 117, 126,
                   135, 144, 153, 162, 171, 180, 189, 198, 207, 216, 225, 234,
                   243, 252, 261, 270, 279, 288, 297, 306, 315, 324, 333, 342,
                   351], dtype=np.float32)
_I_IDS = np.array([50, 150, 250, 350, 450, 550, 650, 750, 850, 950, 1050, 1150,
                   1250, 1350, 1450, 1550, 1650, 1750, 1850, 1950, 2050, 2150,
                   2250, 2350, 2450, 2550, 2650, 2750, 2850, 2950, 3050, 3150,
                   3250, 3350, 3450, 3550, 3650, 3750, 3850, 3950], dtype=np.int64)
_I_ANG = np.array([4, 13, 22, 31, 40, 49, 58, 67, 76, 85, 94, 103, 112, 121,
                   130, 139, 148, 157, 166, 175, 184, 193, 202, 211, 220, 229,
                   238, 247, 256, 265, 274, 283, 292, 301, 310, 319, 328, 337,
                   346, 355], dtype=np.float32)
_MAIN_ANGLES = np.array([0, 45, 90, 135, 180, 225, 270, 315], dtype=np.float32)

_N = 4096
_ROWS = 8 * 2048
_NW = 32
_GPR = _N // 128

_SC_ROWS = 4096
_TC_ROWS = _ROWS - _SC_ROWS
_TRPW = _SC_ROWS * _GPR // _NW
_CH = 256
_SB = 32

_BLK = 1024
_GRID = _TC_ROWS // _BLK


def _build_w() -> np.ndarray:
    rows = []
    for ids, ang in ((_E_IDS, _E_ANG), (_I_IDS, _I_ANG)):
        diff = np.abs(ang[:, None] - _MAIN_ANGLES[None, :])
        min_idx = np.argmin(diff, axis=1)
        closest = _MAIN_ANGLES[min_idx]
        order = np.argsort(closest, kind="stable")
        unique_angles, inv = np.unique(closest[order], return_inverse=True)
        nseg = int(unique_angles.shape[0])
        cnt = np.bincount(inv, minlength=nseg).astype(np.float32)
        sorted_ids = ids[order]
        for b in range(nseg):
            w = np.zeros(_N, np.float32)
            w[sorted_ids[inv == b]] += 1.0 / cnt[b]
            w[ids] -= 1.0 / float(ids.shape[0])
            rows.append(w)
    return np.stack(rows).astype(np.float32)


_W = _build_w()


def _sc_dense_body(table_hbm, out_hbm, dbuf, acc, sem):
    wid = lax.axis_index("s") * 2 + lax.axis_index("c")
    base = _TC_ROWS * _GPR + wid * _TRPW

    zero16 = jnp.zeros((16,), jnp.float32)
    for t in range(_SB):
        for v in range(8):
            acc[t, pl.ds(16 * v, 16)] = zero16

    def _fetch(c, h):
        return pltpu.make_async_copy(
            table_hbm.at[pl.ds(base + c * _CH, _CH)], dbuf.at[h], sem)

    _fetch(0, 0).start()
    n_chunks = _TRPW // _CH

    def _pair(cc, carry):
        for h in range(2):
            c = cc * 2 + h
            _fetch(c, h).wait()

            @pl.when(c < n_chunks - 1)
            def _fire():
                _fetch(c + 1, 1 - h).start()

            def _sub(sb, carry2):
                for t in range(_SB):
                    for v in range(8):
                        x = dbuf[h, sb * _SB + t, pl.ds(16 * v, 16)]
                        plsc.addupdate(acc.at[t, pl.ds(16 * v, 16)], x)
                return carry2

            lax.fori_loop(0, _CH // _SB, _sub, 0)
        return carry

    lax.fori_loop(0, n_chunks // 2, _pair, 0)
    pltpu.sync_copy(acc, out_hbm.at[wid])


@functools.lru_cache(maxsize=1)
def _sc_dense():
    return pl.kernel(
        _sc_dense_body,
        mesh=plsc.VectorSubcoreMesh(core_axis_name="c", subcore_axis_name="s"),
        out_type=jax.ShapeDtypeStruct((_NW, _SB, 128), jnp.float32),
        scratch_types=[
            pltpu.VMEM((2, _CH, 128), jnp.float32),
            pltpu.VMEM((_SB, 128), jnp.float32),
            pltpu.SemaphoreType.DMA,
        ],
        compiler_params=pltpu.CompilerParams(use_tc_tiling_on_sc=True),
    )


def _tc_main_body(x_ref, out_ref, acc_ref):
    i = pl.program_id(0)

    @pl.when(i == 0)
    def _init():
        acc_ref[...] = jnp.zeros_like(acc_ref)

    x = x_ref[...]
    acc_ref[...] += x.reshape(_BLK // 8, 8, _N).sum(axis=0)

    @pl.when(i == _GRID - 1)
    def _fin():
        out_ref[...] = acc_ref[...]


def _tc_finish_body(tc_ref, sc_ref, w_ref, out_ref):
    colsum = tc_ref[...].sum(axis=0, keepdims=True) \
        + sc_ref[...].sum(axis=0, keepdims=True)
    q = (w_ref[...] * colsum).sum(axis=1, keepdims=True)
    scale = 1.0 / (float(_ROWS) * float(_ROWS) * 8.0)
    out_ref[...] = (jnp.sum(q * q, keepdims=True) * scale).reshape(1, 1)


@jax.jit
def kernel(_spikes):
    x2d = _spikes.reshape(_ROWS, _N)
    table = _spikes.reshape(_ROWS * _GPR, 128)

    sc_parts = _sc_dense()(table)

    tc_part = pl.pallas_call(
        _tc_main_body,
        grid=(_GRID,),
        in_specs=[pl.BlockSpec((_BLK, _N), lambda i: (i, 0))],
        out_specs=pl.BlockSpec((8, _N), lambda i: (0, 0)),
        out_shape=jax.ShapeDtypeStruct((8, _N), jnp.float32),
        scratch_shapes=[pltpu.VMEM((8, _N), jnp.float32)],
    )(x2d)

    loss = pl.pallas_call(
        _tc_finish_body,
        in_specs=[
            pl.BlockSpec((8, _N), lambda: (0, 0)),
            pl.BlockSpec((_NW, _N), lambda: (0, 0)),
            pl.BlockSpec((16, _N), lambda: (0, 0)),
        ],
        out_specs=pl.BlockSpec((1, 1), lambda: (0, 0)),
        out_shape=jax.ShapeDtypeStruct((1, 1), jnp.float32),
    )(tc_part, sc_parts.reshape(_NW, _N), jnp.asarray(_W))
    return loss[0, 0]


# speedup vs baseline: 5.8229x; 5.6752x over previous
import functools

import jax
import jax.numpy as jnp
import numpy as np
from jax.experimental import pallas as pl
from jax.experimental.pallas import tpu as pltpu

_E_IDS = np.array([0, 100, 200, 300, 400, 500, 600, 700, 800, 900, 1000, 1100,
                   1200, 1300, 1400, 1500, 1600, 1700, 1800, 1900, 2000, 2100,
                   2200, 2300, 2400, 2500, 2600, 2700, 2800, 2900, 3000, 3100,
                   3200, 3300, 3400, 3500, 3600, 3700, 3800, 3900], dtype=np.int64)
_E_ANG = np.array([0, 9, 18, 27, 36, 45, 54, 63, 72, 81, 90, 99, 108, 117, 126,
                   135, 144, 153, 162, 171, 180, 189, 198, 207, 216, 225, 234,
                   243, 252, 261, 270, 279, 288, 297, 306, 315, 324, 333, 342,
                   351], dtype=np.float32)
_I_IDS = np.array([50, 150, 250, 350, 450, 550, 650, 750, 850, 950, 1050, 1150,
                   1250, 1350, 1450, 1550, 1650, 1750, 1850, 1950, 2050, 2150,
                   2250, 2350, 2450, 2550, 2650, 2750, 2850, 2950, 3050, 3150,
                   3250, 3350, 3450, 3550, 3650, 3750, 3850, 3950], dtype=np.int64)
_I_ANG = np.array([4, 13, 22, 31, 40, 49, 58, 67, 76, 85, 94, 103, 112, 121,
                   130, 139, 148, 157, 166, 175, 184, 193, 202, 211, 220, 229,
                   238, 247, 256, 265, 274, 283, 292, 301, 310, 319, 328, 337,
                   346, 355], dtype=np.float32)
_MAIN_ANGLES = np.array([0, 45, 90, 135, 180, 225, 270, 315], dtype=np.float32)

_N = 4096
_ROWS = 8 * 2048


def _build_proj() -> np.ndarray:
    rows = []
    for ids, ang in ((_E_IDS, _E_ANG), (_I_IDS, _I_ANG)):
        diff = np.abs(ang[:, None] - _MAIN_ANGLES[None, :])
        min_idx = np.argmin(diff, axis=1)
        closest = _MAIN_ANGLES[min_idx]
        order = np.argsort(closest, kind="stable")
        sorted_angles = closest[order]
        unique_angles, inv = np.unique(sorted_angles, return_inverse=True)
        nseg = int(unique_angles.shape[0])
        cnt = np.bincount(inv, minlength=nseg).astype(np.float32)
        sorted_ids = ids[order]
        for b in range(nseg):
            w = np.zeros(_N, np.float32)
            w[sorted_ids[inv == b]] += 1.0 / cnt[b]
            w[ids] -= 1.0 / float(ids.shape[0])
            rows.append(w)
    w = np.stack(rows).astype(np.float32)
    if w.shape[0] % 8:
        w = np.concatenate([w, np.zeros((8 - w.shape[0] % 8, _N), np.float32)])
    return w


_W = _build_proj()

_BLK = 1024
_GRID = _ROWS // _BLK
_NC = 31 * 128


def _loss_body(x_ref, w_ref, out_ref, acc_ref):
    i = pl.program_id(0)

    @pl.when(i == 0)
    def _init():
        acc_ref[...] = jnp.zeros_like(acc_ref)

    x = x_ref[...]
    acc_ref[...] += x.reshape(_BLK // 8, 8, _NC).sum(axis=0)

    @pl.when(i == _GRID - 1)
    def _fin():
        colsum = acc_ref[...].sum(axis=0, keepdims=True)
        q = (w_ref[...] * colsum).sum(axis=1, keepdims=True)
        scale = 1.0 / (float(_ROWS) * float(_ROWS) * 8.0)
        out_ref[...] = (jnp.sum(q * q, keepdims=True) * scale).reshape(1, 1)


@jax.jit
def kernel(_spikes):
    x = _spikes.reshape(_ROWS, _N)
    out = pl.pallas_call(
        _loss_body,
        grid=(_GRID,),
        in_specs=[
            pl.BlockSpec((_BLK, _NC), lambda i: (i, 0)),
            pl.BlockSpec((_W.shape[0], _NC), lambda i: (0, 0)),
        ],
        out_specs=pl.BlockSpec((1, 1), lambda i: (0, 0)),
        out_shape=jax.ShapeDtypeStruct((1, 1), jnp.float32),
        scratch_shapes=[pltpu.VMEM((8, _NC), jnp.float32)],
    )(x, jnp.asarray(_W[:, :_NC]))
    return out[0, 0]


# speedup vs baseline: 5.8888x; 1.0113x over previous
import functools

import jax
import jax.numpy as jnp
import numpy as np
from jax.experimental import pallas as pl
from jax.experimental.pallas import tpu as pltpu

_E_IDS = np.array([0, 100, 200, 300, 400, 500, 600, 700, 800, 900, 1000, 1100,
                   1200, 1300, 1400, 1500, 1600, 1700, 1800, 1900, 2000, 2100,
                   2200, 2300, 2400, 2500, 2600, 2700, 2800, 2900, 3000, 3100,
                   3200, 3300, 3400, 3500, 3600, 3700, 3800, 3900], dtype=np.int64)
_E_ANG = np.array([0, 9, 18, 27, 36, 45, 54, 63, 72, 81, 90, 99, 108, 117, 126,
                   135, 144, 153, 162, 171, 180, 189, 198, 207, 216, 225, 234,
                   243, 252, 261, 270, 279, 288, 297, 306, 315, 324, 333, 342,
                   351], dtype=np.float32)
_I_IDS = np.array([50, 150, 250, 350, 450, 550, 650, 750, 850, 950, 1050, 1150,
                   1250, 1350, 1450, 1550, 1650, 1750, 1850, 1950, 2050, 2150,
                   2250, 2350, 2450, 2550, 2650, 2750, 2850, 2950, 3050, 3150,
                   3250, 3350, 3450, 3550, 3650, 3750, 3850, 3950], dtype=np.int64)
_I_ANG = np.array([4, 13, 22, 31, 40, 49, 58, 67, 76, 85, 94, 103, 112, 121,
                   130, 139, 148, 157, 166, 175, 184, 193, 202, 211, 220, 229,
                   238, 247, 256, 265, 274, 283, 292, 301, 310, 319, 328, 337,
                   346, 355], dtype=np.float32)
_MAIN_ANGLES = np.array([0, 45, 90, 135, 180, 225, 270, 315], dtype=np.float32)

_N = 4096
_ROWS = 8 * 2048


def _build_proj() -> np.ndarray:
    rows = []
    for ids, ang in ((_E_IDS, _E_ANG), (_I_IDS, _I_ANG)):
        diff = np.abs(ang[:, None] - _MAIN_ANGLES[None, :])
        min_idx = np.argmin(diff, axis=1)
        closest = _MAIN_ANGLES[min_idx]
        order = np.argsort(closest, kind="stable")
        sorted_angles = closest[order]
        unique_angles, inv = np.unique(sorted_angles, return_inverse=True)
        nseg = int(unique_angles.shape[0])
        cnt = np.bincount(inv, minlength=nseg).astype(np.float32)
        sorted_ids = ids[order]
        for b in range(nseg):
            w = np.zeros(_N, np.float32)
            w[sorted_ids[inv == b]] += 1.0 / cnt[b]
            w[ids] -= 1.0 / float(ids.shape[0])
            rows.append(w)
    w = np.stack(rows).astype(np.float32)
    if w.shape[0] % 8:
        w = np.concatenate([w, np.zeros((8 - w.shape[0] % 8, _N), np.float32)])
    return w


_W = _build_proj()

_BLK = 512
_GRID = _ROWS // _BLK
_NC = 31 * 128


def _loss_body(x_ref, w_ref, out_ref, acc_ref):
    i = pl.program_id(0)

    @pl.when(i == 0)
    def _init():
        acc_ref[...] = jnp.zeros_like(acc_ref)

    x = x_ref[...]
    acc_ref[...] += x.reshape(_BLK // 8, 8, _NC).sum(axis=0)

    @pl.when(i == _GRID - 1)
    def _fin():
        colsum = acc_ref[...].sum(axis=0, keepdims=True)
        q = (w_ref[...] * colsum).sum(axis=1, keepdims=True)
        scale = 1.0 / (float(_ROWS) * float(_ROWS) * 8.0)
        out_ref[...] = (jnp.sum(q * q, keepdims=True) * scale).reshape(1, 1)


@jax.jit
def kernel(_spikes):
    x = _spikes.reshape(_ROWS, _N)
    out = pl.pallas_call(
        _loss_body,
        grid=(_GRID,),
        in_specs=[
            pl.BlockSpec((_BLK, _NC), lambda i: (i, 0)),
            pl.BlockSpec((_W.shape[0], _NC), lambda i: (0, 0)),
        ],
        out_specs=pl.BlockSpec((1, 1), lambda i: (0, 0)),
        out_shape=jax.ShapeDtypeStruct((1, 1), jnp.float32),
        scratch_shapes=[pltpu.VMEM((8, _NC), jnp.float32)],
    )(x, jnp.asarray(_W[:, :_NC]))
    return out[0, 0]
